# baseline (speedup 1.0000x reference)
import jax
import jax.numpy as jnp
from jax import lax
from jax.experimental import pallas as pl
from jax.experimental.pallas import tpu as pltpu
from jax.experimental.pallas import tpu_sc as plsc

EPS_DIV = 1e-06
LANES = 16
NUM_WORKERS = 32
ROW_ELEMS = 128


def _sc_recip_gather(scale_g, idx, col_base, col_len, out_w):
    g = idx.shape[0]
    b_per_w = ((col_len + NUM_WORKERS - 1) // NUM_WORKERS + ROW_ELEMS - 1) \
        // ROW_ELEMS * ROW_ELEMS
    n_rows = b_per_w // ROW_ELEMS
    full_workers = col_len // b_per_w
    tail_base = full_workers * b_per_w
    tail_len = col_len - tail_base
    tail_full_rows = tail_len // ROW_ELEMS
    tail_rem = tail_len - tail_full_rows * ROW_ELEMS
    tail_out = (tail_len + ROW_ELEMS - 1) // ROW_ELEMS * ROW_ELEMS
    assert full_workers + (1 if tail_len else 0) == NUM_WORKERS
    mesh = plsc.VectorSubcoreMesh(core_axis_name="c", subcore_axis_name="s")

    def body(scale_hbm, idx_hbm, out_hbm, idx_v, vals_v, sem_i, sem_g):
        nc = lax.axis_size("c")
        wid = lax.axis_index("s") * nc + lax.axis_index("c")
        is_tail = wid == NUM_WORKERS - 1
        base = wid * b_per_w

        if tail_len:
            @pl.when(~is_tail)
            def _():
                cps = [pltpu.async_copy(
                    idx_hbm.at[pl.ds(col_base + base + j * ROW_ELEMS, ROW_ELEMS)],
                    idx_v.at[j], sem_i) for j in range(n_rows)]
                for cp in cps:
                    cp.wait()

            @pl.when(is_tail)
            def _():
                cps = [pltpu.async_copy(
                    idx_hbm.at[pl.ds(col_base + tail_base + j * ROW_ELEMS,
                                     ROW_ELEMS)],
                    idx_v.at[j], sem_i) for j in range(tail_full_rows)]
                if tail_rem:
                    cps.append(pltpu.async_copy(
                        idx_hbm.at[pl.ds(
                            col_base + tail_base + tail_full_rows * ROW_ELEMS,
                            tail_rem)],
                        idx_v.at[tail_full_rows, pl.ds(0, tail_rem)], sem_i))
                for cp in cps:
                    cp.wait()
                for j in range(tail_full_rows, n_rows):
                    for v in range(ROW_ELEMS // LANES):
                        off = v * LANES
                        iv = idx_v[j, pl.ds(off, LANES)]
                        idx_v[j, pl.ds(off, LANES)] = jnp.minimum(
                            jnp.maximum(iv, 0), g - 1)
        else:
            cps = [pltpu.async_copy(
                idx_hbm.at[pl.ds(col_base + base + j * ROW_ELEMS, ROW_ELEMS)],
                idx_v.at[j], sem_i) for j in range(n_rows)]
            for cp in cps:
                cp.wait()

        cps = [pltpu.async_copy(scale_hbm.at[idx_v.at[j]],
                                vals_v.at[pl.ds(j * ROW_ELEMS, ROW_ELEMS)],
                                sem_g)
               for j in range(n_rows)]
        for cp in cps:
            cp.wait()

        for i in range(b_per_w // LANES):
            off = i * LANES
            v = vals_v[pl.ds(off, LANES)]
            vals_v[pl.ds(off, LANES)] = 1.0 / (v + EPS_DIV)

        if tail_len:
            @pl.when(~is_tail)
            def _():
                pltpu.sync_copy(vals_v.at[pl.ds(0, b_per_w)],
                                out_hbm.at[0, pl.ds(base, b_per_w)])

            @pl.when(is_tail)
            def _():
                pltpu.sync_copy(vals_v.at[pl.ds(0, tail_out)],
                                out_hbm.at[0, pl.ds(tail_base, tail_out)])
        else:
            pltpu.sync_copy(vals_v.at[pl.ds(0, b_per_w)],
                            out_hbm.at[0, pl.ds(base, b_per_w)])

    fn = pl.kernel(
        body,
        out_type=jax.ShapeDtypeStruct((1, out_w), jnp.float32),
        mesh=mesh,
        compiler_params=pltpu.CompilerParams(needs_layout_passes=False),
        scratch_types=[
            pltpu.VMEM((n_rows, ROW_ELEMS), jnp.int32),
            pltpu.VMEM((b_per_w,), jnp.float32),
            pltpu.SemaphoreType.DMA,
            pltpu.SemaphoreType.DMA,
        ],
    )
    return fn(scale_g, idx)


def _tc_scale_mul(x_ng, recip_row, block_rows, block_cols, col_block,
                  prev_out=None):
    n, g = x_ng.shape

    if prev_out is None:
        def body(x_ref, r_ref, o_ref):
            o_ref[...] = x_ref[...] * r_ref[...]

        operands = (x_ng, recip_row)
        extra_specs = []
        aliases = {}
    else:
        def body(x_ref, r_ref, _prev_ref, o_ref):
            o_ref[...] = x_ref[...] * r_ref[...]

        operands = (x_ng, recip_row, prev_out)
        extra_specs = [pl.BlockSpec(memory_space=pl.ANY)]
        aliases = {2: 0}

    return pl.pallas_call(
        body,
        grid=(pl.cdiv(n, block_rows),),
        in_specs=[
            pl.BlockSpec((block_rows, block_cols), lambda i: (i, col_block)),
            pl.BlockSpec((1, block_cols), lambda i: (0, 0)),
        ] + extra_specs,
        out_specs=pl.BlockSpec((block_rows, block_cols),
                               lambda i: (i, col_block)),
        out_shape=jax.ShapeDtypeStruct((n, g), jnp.float32),
        input_output_aliases=aliases,
        compiler_params=pltpu.CompilerParams(vmem_limit_bytes=128 * 1024 * 1024),
    )(*operands)


@jax.jit
def kernel(x_ng, scale_g, idx):
    n, g = x_ng.shape
    split = 20480
    recip_a = _sc_recip_gather(scale_g, idx, 0, split, split)
    recip_b = _sc_recip_gather(scale_g, idx, split, g - split, split)
    out_a = _tc_scale_mul(x_ng, recip_a, 112, split, 0)
    return _tc_scale_mul(x_ng, recip_b, 112, split, 1, prev_out=out_a)


# speedup vs baseline: 1.0113x; 1.0113x over previous
import jax
import jax.numpy as jnp
from jax import lax
from jax.experimental import pallas as pl
from jax.experimental.pallas import tpu as pltpu
from jax.experimental.pallas import tpu_sc as plsc

EPS_DIV = 1e-06
LANES = 16
NUM_WORKERS = 32


def _sc_recip_gather(scale_g, idx, g_pad):
    g = idx.shape[0]
    b_per_w = g_pad // NUM_WORKERS
    tail_base = (NUM_WORKERS - 1) * b_per_w
    tail_len = g - tail_base
    tail_out = ((tail_len + 127) // 128) * 128
    mesh = plsc.VectorSubcoreMesh(core_axis_name="c", subcore_axis_name="s")

    n_rows = b_per_w // 128
    tail_full_rows = tail_len // 128

    def body(scale_hbm, idx_hbm, out_hbm, idx_v, vals_v, out_v, sem_i, sem_g):
        nc = lax.axis_size("c")
        wid = lax.axis_index("s") * nc + lax.axis_index("c")
        is_tail = wid == NUM_WORKERS - 1
        base = wid * b_per_w

        @pl.when(~is_tail)
        def _():
            cps = [pltpu.async_copy(idx_hbm.at[pl.ds(base + j * 128, 128)],
                                    idx_v.at[j], sem_i)
                   for j in range(n_rows)]
            for cp in cps:
                cp.wait()

        @pl.when(is_tail)
        def _():
            rem = tail_len - tail_full_rows * 128
            cps = [pltpu.async_copy(idx_hbm.at[pl.ds(tail_base + j * 128, 128)],
                                    idx_v.at[j], sem_i)
                   for j in range(tail_full_rows)]
            cps.append(pltpu.async_copy(
                idx_hbm.at[pl.ds(tail_base + tail_full_rows * 128, rem)],
                idx_v.at[tail_full_rows, pl.ds(0, rem)], sem_i))
            for cp in cps:
                cp.wait()
            for j in range(tail_full_rows, n_rows):
                for v in range(128 // LANES):
                    off = v * LANES
                    iv = idx_v[j, pl.ds(off, LANES)]
                    idx_v[j, pl.ds(off, LANES)] = jnp.minimum(
                        jnp.maximum(iv, 0), g - 1)

        cps = [pltpu.async_copy(scale_hbm.at[idx_v.at[j]],
                                vals_v.at[pl.ds(j * 128, 128)], sem_g)
               for j in range(n_rows)]
        for cp in cps:
            cp.wait()

        for i in range(b_per_w // LANES):
            off = i * LANES
            v = vals_v[pl.ds(off, LANES)]
            out_v[pl.ds(off, LANES)] = 1.0 / (v + EPS_DIV)

        @pl.when(~is_tail)
        def _():
            pltpu.sync_copy(out_v.at[pl.ds(0, b_per_w)],
                            out_hbm.at[0, pl.ds(base, b_per_w)])

        @pl.when(is_tail)
        def _():
            pltpu.sync_copy(out_v.at[pl.ds(0, tail_out)],
                            out_hbm.at[0, pl.ds(tail_base, tail_out)])

    fn = pl.kernel(
        body,
        out_type=jax.ShapeDtypeStruct((1, g_pad), jnp.float32),
        mesh=mesh,
        compiler_params=pltpu.CompilerParams(needs_layout_passes=False),
        scratch_types=[
            pltpu.VMEM((n_rows, 128), jnp.int32),
            pltpu.VMEM((b_per_w,), jnp.float32),
            pltpu.VMEM((b_per_w,), jnp.float32),
            pltpu.SemaphoreType.DMA,
            pltpu.SemaphoreType.DMA,
        ],
    )
    return fn(scale_g, idx)


def _tc_scale_mul(x_ng, recip_row, block_rows):
    n, g = x_ng.shape
    g_pad = recip_row.shape[1]

    def body(x_ref, r_ref, o_ref):
        o_ref[...] = x_ref[...] * r_ref[...][:, :g]

    return pl.pallas_call(
        body,
        grid=(pl.cdiv(n, block_rows),),
        in_specs=[
            pl.BlockSpec((block_rows, g), lambda i: (i, 0)),
            pl.BlockSpec((1, g_pad), lambda i: (0, 0)),
        ],
        out_specs=pl.BlockSpec((block_rows, g), lambda i: (i, 0)),
        out_shape=jax.ShapeDtypeStruct((n, g), jnp.float32),
        compiler_params=pltpu.CompilerParams(vmem_limit_bytes=128 * 1024 * 1024),
    )(x_ng, recip_row)


@jax.jit
def kernel(x_ng, scale_g, idx):
    n, g = x_ng.shape
    chunk = NUM_WORKERS * LANES
    g_pad = ((g + chunk - 1) // chunk) * chunk
    recip_row = _sc_recip_gather(scale_g, idx, g_pad)
    return _tc_scale_mul(x_ng, recip_row, block_rows=112)
